# Initial kernel scaffold; baseline (speedup 1.0000x reference)
import jax
import jax.numpy as jnp
from jax.experimental import pallas as pl


def kernel(input_ids, table):
    raise NotImplementedError("write your pallas kernel here")


# speedup vs baseline: 1.5688x; 1.5688x over previous
import functools

import jax
import jax.numpy as jnp
from jax import lax
from jax.experimental import pallas as pl
from jax.experimental.pallas import tpu as pltpu
from jax.experimental.pallas import tpu_sc as plsc

NC = 2
NS = 16
NW = NC * NS

CHUNK = 128
CHUNKS_PER_BLK = 13
BLK = CHUNK * CHUNKS_PER_BLK


@functools.partial(jax.jit, static_argnames=())
def _gather_rows(idx2, table):
    n_chunks = idx2.shape[0]
    D = table.shape[1]
    B = n_chunks * CHUNK
    chunks_per_w = n_chunks // NW
    n_blocks = chunks_per_w // CHUNKS_PER_BLK

    mesh = plsc.VectorSubcoreMesh(core_axis_name="c", subcore_axis_name="s")

    @functools.partial(
        pl.kernel,
        mesh=mesh,
        out_type=jax.ShapeDtypeStruct((B, D), jnp.float32),
        scratch_types=[
            pltpu.VMEM((chunks_per_w, CHUNK), jnp.int32),
            pltpu.VMEM((BLK, D), jnp.float32),
            pltpu.SemaphoreType.DMA,
        ],
        compiler_params=pltpu.CompilerParams(use_tc_tiling_on_sc=False),
    )
    def k(idx_hbm, table_hbm, out_hbm, idx_v, rows_v, sem):
        wid = lax.axis_index("s") * NC + lax.axis_index("c")
        chunk_base = wid * chunks_per_w
        out_base = chunk_base * CHUNK
        pltpu.sync_copy(idx_hbm.at[pl.ds(chunk_base, chunks_per_w)], idx_v)

        def body(blk, _):
            copies = []
            for j in range(CHUNKS_PER_BLK):
                cj = blk * CHUNKS_PER_BLK + j
                copies.append(
                    pltpu.async_copy(
                        table_hbm.at[idx_v.at[cj]],
                        rows_v.at[pl.ds(j * CHUNK, CHUNK)],
                        sem,
                    )
                )
            for c in copies:
                c.wait()
            pltpu.sync_copy(rows_v, out_hbm.at[pl.ds(out_base + blk * BLK, BLK)])
            return ()

        lax.fori_loop(0, n_blocks, body, (), unroll=False)

    return k(idx2, table)


def kernel(input_ids, table):
    S, F = input_ids.shape
    D = table.shape[1]
    idx = input_ids.reshape(-1).astype(jnp.int32)
    idx2 = idx.reshape(-1, CHUNK)
    out = _gather_rows(idx2, table)
    return out.reshape(S, F, D)
